# baseline (speedup 1.0000x reference)
import jax
import jax.numpy as jnp
from jax.experimental import pallas as pl
from jax.experimental.pallas import tpu as pltpu

D = 768
P, Q, S = 7, 1, 69
N = 5
ROW = (P + Q + S) * D


def _concat_body(p_ref, q_ref, s_ref, o_ref, sem_p, sem_q, sem_s):
    cp = pltpu.make_async_copy(p_ref, o_ref.at[:, : P * D], sem_p)
    cq = pltpu.make_async_copy(q_ref, o_ref.at[:, P * D : (P + Q) * D], sem_q)
    cs = pltpu.make_async_copy(s_ref, o_ref.at[:, (P + Q) * D :], sem_s)
    cp.start()
    cq.start()
    cs.start()
    cp.wait()
    cq.wait()
    cs.wait()


def kernel(embedding_prefix, learnable_quality, embedding_suffix):
    p = embedding_prefix.reshape(N, P * D)
    q = learnable_quality.reshape(N, Q * D)
    s = embedding_suffix.reshape(N, S * D)
    out = pl.pallas_call(
        _concat_body,
        out_shape=jax.ShapeDtypeStruct((N, ROW), jnp.float32),
        in_specs=[
            pl.BlockSpec(memory_space=pl.ANY),
            pl.BlockSpec(memory_space=pl.ANY),
            pl.BlockSpec(memory_space=pl.ANY),
        ],
        out_specs=pl.BlockSpec(memory_space=pl.ANY),
        scratch_shapes=[
            pltpu.SemaphoreType.DMA,
            pltpu.SemaphoreType.DMA,
            pltpu.SemaphoreType.DMA,
        ],
    )(p, q, s)
    return out.reshape(N, P + Q + S, D)


# speedup vs baseline: 26.4959x; 26.4959x over previous
import jax
import jax.numpy as jnp
from jax.experimental import pallas as pl

D = 768
P, Q, S = 7, 1, 69
N = 5


def _concat_body(p_ref, q_ref, s_ref, o_ref):
    o_ref[:, :P, :] = p_ref[...]
    o_ref[:, P : P + Q, :] = q_ref[...][:, None, :]
    o_ref[:, P + Q :, :] = s_ref[...]


def kernel(embedding_prefix, learnable_quality, embedding_suffix):
    return pl.pallas_call(
        _concat_body,
        out_shape=jax.ShapeDtypeStruct((N, P + Q + S, D), jnp.float32),
    )(embedding_prefix, learnable_quality, embedding_suffix)
